# baseline (speedup 1.0000x reference)
import math

import jax
import jax.numpy as jnp
from jax import lax
from jax.experimental import pallas as pl
from jax.experimental.pallas import tpu as pltpu
from jax.experimental.pallas import tpu_sc as plsc

D_MODEL = 64
SCALE = math.sqrt(D_MODEL)

NC = 2
NS = 16
NW = NC * NS
LANES = 16

CHUNK = 512
GATHER = 128
G_PER_CHUNK = CHUNK // GATHER


def _build(B):
    assert B % (NW * CHUNK) == 0
    b_per_w = B // NW
    n_chunks = b_per_w // CHUNK
    mesh = plsc.VectorSubcoreMesh(
        core_axis_name="c", subcore_axis_name="s",
        num_cores=NC, num_subcores=NS)

    def body(x_hbm, table_hbm, out_hbm, idx_v, rows_v, sem):
        wid = lax.axis_index("s") * NC + lax.axis_index("c")
        base = wid * b_per_w

        def chunk_body(g, carry):
            off = base + g * CHUNK
            pltpu.sync_copy(x_hbm.at[pl.ds(off, CHUNK)], idx_v)
            copies = [
                pltpu.async_copy(
                    table_hbm.at[idx_v.at[pl.ds(j * GATHER, GATHER)]],
                    rows_v.at[pl.ds(j * GATHER, GATHER)],
                    sem)
                for j in range(G_PER_CHUNK)
            ]
            for c in copies:
                c.wait()

            def scale_rows(r, carry2):
                for u in range(4):
                    row = r * 4 + u
                    for col in range(D_MODEL // LANES):
                        sl = pl.ds(col * LANES, LANES)
                        rows_v[row, sl] = rows_v[row, sl] * SCALE
                return carry2

            lax.fori_loop(0, CHUNK // 4, scale_rows, 0)
            pltpu.sync_copy(rows_v, out_hbm.at[pl.ds(off, CHUNK)])
            return carry

        lax.fori_loop(0, n_chunks, chunk_body, 0)

    kern = pl.kernel(
        body,
        out_type=jax.ShapeDtypeStruct((B, D_MODEL), jnp.float32),
        mesh=mesh,
        scratch_types=[
            pltpu.VMEM((CHUNK,), jnp.int32),
            pltpu.VMEM((CHUNK, D_MODEL), jnp.float32),
            pltpu.SemaphoreType.DMA,
        ],
        compiler_params=pltpu.CompilerParams(use_tc_tiling_on_sc=False),
    )
    return kern


def kernel(x, table):
    B0, B1 = x.shape
    flat = x.reshape(B0 * B1).astype(jnp.int32)
    out = _build(B0 * B1)(flat, table)
    return out.reshape(B0, B1, D_MODEL)


# speedup vs baseline: 1.0923x; 1.0923x over previous
import math

import jax
import jax.numpy as jnp
from jax import lax
from jax.experimental import pallas as pl
from jax.experimental.pallas import tpu as pltpu
from jax.experimental.pallas import tpu_sc as plsc

D_MODEL = 64
SCALE = math.sqrt(D_MODEL)

NC = 2
NS = 16
NW = NC * NS
LANES = 16

CHUNK = 512
GATHER = 128
G_PER_CHUNK = CHUNK // GATHER
NBUF = 2


def _build(B):
    assert B % (NW * CHUNK * NBUF) == 0
    b_per_w = B // NW
    n_chunks = b_per_w // CHUNK
    mesh = plsc.VectorSubcoreMesh(
        core_axis_name="c", subcore_axis_name="s",
        num_cores=NC, num_subcores=NS)

    def body(x_hbm, table_hbm, out_hbm, idx_v, rows0, rows1, sem_g0, sem_g1,
             sem_w0, sem_w1):
        rows = (rows0, rows1)
        sem_g = (sem_g0, sem_g1)
        sem_w = (sem_w0, sem_w1)
        wid = lax.axis_index("s") * NC + lax.axis_index("c")
        base = wid * b_per_w

        pltpu.sync_copy(x_hbm.at[pl.ds(base, b_per_w)], idx_v)

        def fire_gather(g, b):
            for j in range(G_PER_CHUNK):
                pltpu.async_copy(
                    table_hbm.at[idx_v.at[pl.ds(g * CHUNK + j * GATHER,
                                                GATHER)]],
                    rows[b].at[pl.ds(j * GATHER, GATHER)],
                    sem_g[b])

        def wait_gather(b):
            for j in range(G_PER_CHUNK):
                pltpu.make_async_copy(
                    table_hbm.at[idx_v.at[pl.ds(j * GATHER, GATHER)]],
                    rows[b].at[pl.ds(j * GATHER, GATHER)],
                    sem_g[b]).wait()

        def fire_wb(g, b):
            pltpu.async_copy(
                rows[b], out_hbm.at[pl.ds(base + g * CHUNK, CHUNK)], sem_w[b])

        def wait_wb(b):
            pltpu.make_async_copy(
                rows[b], out_hbm.at[pl.ds(0, CHUNK)], sem_w[b]).wait()

        def scale(b):
            buf = rows[b]

            @plsc.parallel_loop(0, CHUNK, unroll=8)
            def _(r):
                for col in range(D_MODEL // LANES):
                    sl = pl.ds(col * LANES, LANES)
                    buf[r, sl] = buf[r, sl] * SCALE

        for b in range(NBUF):
            fire_gather(b, b)

        def pair_body(g2, carry):
            for b in range(NBUF):
                g = g2 + b
                wait_gather(b)
                scale(b)
                fire_wb(g, b)

                @pl.when(g + NBUF < n_chunks)
                def _():
                    wait_wb(b)
                    fire_gather(g + NBUF, b)
                return_val = carry
            return return_val

        lax.fori_loop(0, n_chunks // NBUF, lambda i, c: pair_body(i * NBUF, c),
                      0)
        for b in range(NBUF):
            wait_wb(b)

    kern = pl.kernel(
        body,
        out_type=jax.ShapeDtypeStruct((B, D_MODEL), jnp.float32),
        mesh=mesh,
        scratch_types=[
            pltpu.VMEM((B // NW,), jnp.int32),
            pltpu.VMEM((CHUNK, D_MODEL), jnp.float32),
            pltpu.VMEM((CHUNK, D_MODEL), jnp.float32),
            pltpu.SemaphoreType.DMA,
            pltpu.SemaphoreType.DMA,
            pltpu.SemaphoreType.DMA,
            pltpu.SemaphoreType.DMA,
        ],
        compiler_params=pltpu.CompilerParams(use_tc_tiling_on_sc=False),
    )
    return kern


def kernel(x, table):
    B0, B1 = x.shape
    flat = x.reshape(B0 * B1).astype(jnp.int32)
    out = _build(B0 * B1)(flat, table)
    return out.reshape(B0, B1, D_MODEL)
